# baseline (speedup 1.0000x reference)
import jax
import jax.numpy as jnp
from jax.experimental import pallas as pl

N_NODES = 10000
NUM_GRAPHS = 32
HIDDEN = 64


def _gcn_conv(x, edge_index, W, b, n_nodes):
    loop = jnp.arange(n_nodes, dtype=edge_index.dtype)
    src = jnp.concatenate([edge_index[0], loop])
    dst = jnp.concatenate([edge_index[1], loop])
    h = x @ W
    deg = jnp.zeros((n_nodes,), dtype=x.dtype).at[dst].add(1.0)
    dinv = jax.lax.rsqrt(deg)
    norm = dinv[src] * dinv[dst]
    msg = h[src] * norm[:, None]
    out = jnp.zeros((n_nodes, W.shape[1]), dtype=x.dtype).at[dst].add(msg)
    return out + b


def _batch_norm(h, gamma, beta, eps=1e-5):
    m = jnp.mean(h, axis=0)
    v = jnp.var(h, axis=0)
    return (h - m) * jax.lax.rsqrt(v + eps) * gamma + beta


def _encoder(x, edge_index, batch, W1, b1, g1, be1, W2, b2, g2, be2):
    n = x.shape[0]
    h = _gcn_conv(x, edge_index, W1, b1, n)
    h = jax.nn.relu(_batch_norm(h, g1, be1))
    h = _gcn_conv(h, edge_index, W2, b2, n)
    h = jax.nn.relu(_batch_norm(h, g2, be2))
    s = jax.ops.segment_sum(h, batch, num_segments=NUM_GRAPHS)
    cnt = jax.ops.segment_sum(jnp.ones((n,), dtype=h.dtype), batch, num_segments=NUM_GRAPHS)
    return s / jnp.maximum(cnt, 1.0)[:, None]


def _head_kernel(h_ref, w1_ref, b1_ref, w2_ref, b2_ref, w3_ref, b3_ref, o_ref):
    h = h_ref[...]
    h = jax.nn.relu(h @ w1_ref[...] + b1_ref[...])
    h = jax.nn.relu(h @ w2_ref[...] + b2_ref[...])
    o_ref[...] = h @ w3_ref[...] + b3_ref[...]


def kernel(block_x, tx_x, b_W1, b_b1, b_g1, b_be1, b_W2, b_b2, b_g2, b_be2,
           t_W1, t_b1, t_g1, t_be1, t_W2, t_b2, t_g2, t_be2,
           fc1_W, fc1_b, fc2_W, fc2_b, fc3_W, fc3_b,
           block_edge_index, block_batch, tx_edge_index, tx_batch):
    block_emb = _encoder(block_x, block_edge_index, block_batch,
                         b_W1, b_b1, b_g1, b_be1, b_W2, b_b2, b_g2, b_be2)
    tx_emb = _encoder(tx_x, tx_edge_index, tx_batch,
                      t_W1, t_b1, t_g1, t_be1, t_W2, t_b2, t_g2, t_be2)
    h = jnp.concatenate([block_emb, tx_emb], axis=1)
    out = pl.pallas_call(
        _head_kernel,
        out_shape=jax.ShapeDtypeStruct((NUM_GRAPHS, 2), jnp.float32),
    )(h, fc1_W, fc1_b[None, :], fc2_W, fc2_b[None, :], fc3_W, fc3_b[None, :])
    return out


# speedup vs baseline: 6.5289x; 6.5289x over previous
import functools

import jax
import jax.numpy as jnp
from jax import lax
from jax.experimental import pallas as pl
from jax.experimental.pallas import tpu as pltpu
from jax.experimental.pallas import tpu_sc as plsc

N_NODES = 10000
NUM_GRAPHS = 32
HIDDEN = 64
LW = 128

NN = 2 * N_NODES
NE = 2 * 320000
NG = 2 * NUM_GRAPHS

NC = 2
NS = 16
HALF = 10240
TRASH = HALF
ACCR = HALF + LW
RPT = ACCR // NS
WPT = HALF // NS

EPT = NE // NS
CH = 128
NFULL = EPT // CH
TAIL = EPT - NFULL * CH

_mesh = plsc.VectorSubcoreMesh(core_axis_name="c", subcore_axis_name="s",
                               num_cores=NC, num_subcores=NS)


def _fill_rows(ref, nrows, value):
    vec = jnp.full((16,), value, jnp.float32)

    def body(r, _):
        for cc in range(LW // 16):
            ref[r, pl.ds(cc * 16, 16)] = vec
        return 0

    lax.fori_loop(0, nrows, body, 0)


def _zero_accum(rows_v, acc_sh, s):
    row0 = s * RPT
    for j in range(RPT // CH):
        pltpu.sync_copy(rows_v, acc_sh.at[pl.ds(row0 + j * CH, CH)])
    rem = RPT - (RPT // CH) * CH
    if rem:
        pltpu.sync_copy(rows_v.at[pl.ds(0, rem)],
                        acc_sh.at[pl.ds(row0 + (RPT // CH) * CH, rem)])


def _localize(dst_v, dstl_v, n, base):
    for j in range(n // 16):
        d = dst_v[pl.ds(j * 16, 16)] - base
        ok = (d >= 0) & (d < HALF)
        dstl_v[pl.ds(j * 16, 16)] = jnp.where(ok, d, TRASH)


def _writeout(acc_sh, out_hbm, c, s):
    for j in range(WPT // CH):
        r = s * WPT + j * CH
        pltpu.sync_copy(acc_sh.at[pl.ds(r, CH)], out_hbm.at[c, pl.ds(r, CH)])


@functools.partial(
    pl.kernel,
    out_type=jax.ShapeDtypeStruct((NC, HALF, LW), jnp.float32),
    mesh=_mesh,
    scratch_types=[
        pltpu.VMEM((CH,), jnp.int32),
        pltpu.VMEM((TAIL,), jnp.int32),
        pltpu.VMEM((CH,), jnp.int32),
        pltpu.VMEM((TAIL,), jnp.int32),
        pltpu.VMEM((CH,), jnp.int32),
        pltpu.VMEM((TAIL,), jnp.int32),
        pltpu.VMEM((CH, LW), jnp.float32),
        pltpu.VMEM_SHARED((ACCR, LW), jnp.float32),
        pltpu.SemaphoreType.DMA,
    ],
)
def _scat_kernel(hs_hbm, src_hbm, dst_hbm, out_hbm,
                 src_v, src_t, dst_v, dst_t, dstl_v, dstl_t, rows_v, acc_sh,
                 gsem):
    c = lax.axis_index("c")
    s = lax.axis_index("s")
    base = c * HALF
    _fill_rows(rows_v, CH, 0.0)
    _zero_accum(rows_v, acc_sh, s)
    plsc.subcore_barrier()

    ebase = s * EPT

    def body(i, _):
        off = pl.multiple_of(ebase + i * CH, 8)
        pltpu.sync_copy(src_hbm.at[pl.ds(off, CH)], src_v)
        pltpu.sync_copy(dst_hbm.at[pl.ds(off, CH)], dst_v)
        _localize(dst_v, dstl_v, CH, base)
        pltpu.async_copy(hs_hbm.at[src_v], rows_v, gsem).wait()
        pltpu.sync_copy(rows_v, acc_sh.at[dstl_v], add=True)
        return 0

    lax.fori_loop(0, NFULL, body, 0)

    offt = pl.multiple_of(ebase + NFULL * CH, 8)
    pltpu.sync_copy(src_hbm.at[pl.ds(offt, TAIL)], src_t)
    pltpu.sync_copy(dst_hbm.at[pl.ds(offt, TAIL)], dst_t)
    _localize(dst_t, dstl_t, TAIL, base)
    pltpu.async_copy(hs_hbm.at[src_t], rows_v.at[pl.ds(0, TAIL)], gsem).wait()
    pltpu.sync_copy(rows_v.at[pl.ds(0, TAIL)], acc_sh.at[dstl_t], add=True)

    plsc.subcore_barrier()
    _writeout(acc_sh, out_hbm, c, s)


@functools.partial(
    pl.kernel,
    out_type=jax.ShapeDtypeStruct((NC, HALF, LW), jnp.float32),
    mesh=_mesh,
    scratch_types=[
        pltpu.VMEM((CH,), jnp.int32),
        pltpu.VMEM((TAIL,), jnp.int32),
        pltpu.VMEM((CH,), jnp.int32),
        pltpu.VMEM((TAIL,), jnp.int32),
        pltpu.VMEM((CH, LW), jnp.float32),
        pltpu.VMEM((CH, LW), jnp.float32),
        pltpu.VMEM_SHARED((ACCR, LW), jnp.float32),
    ],
)
def _deg_kernel(dst_hbm, out_hbm, dst_v, dst_t, dstl_v, dstl_t, zer_v, ones_v,
                acc_sh):
    c = lax.axis_index("c")
    s = lax.axis_index("s")
    base = c * HALF
    _fill_rows(zer_v, CH, 0.0)
    _fill_rows(ones_v, CH, 1.0)
    _zero_accum(zer_v, acc_sh, s)
    plsc.subcore_barrier()

    ebase = s * EPT

    def body(i, _):
        off = pl.multiple_of(ebase + i * CH, 8)
        pltpu.sync_copy(dst_hbm.at[pl.ds(off, CH)], dst_v)
        _localize(dst_v, dstl_v, CH, base)
        pltpu.sync_copy(ones_v, acc_sh.at[dstl_v], add=True)
        return 0

    lax.fori_loop(0, NFULL, body, 0)

    offt = pl.multiple_of(ebase + NFULL * CH, 8)
    pltpu.sync_copy(dst_hbm.at[pl.ds(offt, TAIL)], dst_t)
    _localize(dst_t, dstl_t, TAIL, base)
    pltpu.sync_copy(ones_v.at[pl.ds(0, TAIL)], acc_sh.at[dstl_t], add=True)

    plsc.subcore_barrier()
    _writeout(acc_sh, out_hbm, c, s)


def _unsplit(parts_ref):
    full = jnp.concatenate([parts_ref[0], parts_ref[1]], axis=0)
    return full[:NN, :HIDDEN]


def _k1_body(bx_ref, tx_ref, bw_ref, tw_ref, degp_ref, hs_ref, dinv_ref):
    hb = jnp.dot(bx_ref[...], bw_ref[...], preferred_element_type=jnp.float32)
    ht = jnp.dot(tx_ref[...], tw_ref[...], preferred_element_type=jnp.float32)
    h = jnp.concatenate([hb, ht], axis=0)
    dinv = lax.rsqrt(_unsplit(degp_ref) + 1.0)
    hs = h * dinv
    hs_ref[...] = jnp.concatenate([hs, jnp.zeros_like(hs)], axis=1)
    dinv_ref[...] = dinv


def _combine_bn_relu(acc_ref, hs_ref, dinv_ref, b_b, t_b, b_g, t_g, b_be, t_be):
    acc = _unsplit(acc_ref)
    dinv = dinv_ref[...]
    hs = hs_ref[...][:, :HIDDEN]
    z = dinv * acc + dinv * hs
    zb = z[:N_NODES] + b_b[...]
    zt = z[N_NODES:] + t_b[...]

    def bn_relu(y, g, be):
        m = jnp.mean(y, axis=0, keepdims=True)
        v = jnp.mean((y - m) * (y - m), axis=0, keepdims=True)
        return jax.nn.relu((y - m) * lax.rsqrt(v + 1e-5) * g + be)

    return bn_relu(zb, b_g[...], b_be[...]), bn_relu(zt, t_g[...], t_be[...])


def _k2a_body(acc_ref, hs_ref, dinv_ref, bb_ref, tb_ref, bg_ref, tg_ref,
              bbe_ref, tbe_ref, x2_ref):
    xb, xt = _combine_bn_relu(acc_ref, hs_ref, dinv_ref, bb_ref, tb_ref,
                              bg_ref, tg_ref, bbe_ref, tbe_ref)
    x2_ref[...] = jnp.concatenate([xb, xt], axis=0)


def _k2b_body(x2_ref, bw2_ref, tw2_ref, dinv_ref, hs2_ref):
    x2 = x2_ref[...]
    hb = jnp.dot(x2[:N_NODES], bw2_ref[...],
                 preferred_element_type=jnp.float32)
    ht = jnp.dot(x2[N_NODES:], tw2_ref[...],
                 preferred_element_type=jnp.float32)
    h2 = jnp.concatenate([hb, ht], axis=0)
    hs2 = h2 * dinv_ref[...]
    hs2_ref[...] = jnp.concatenate([hs2, jnp.zeros_like(hs2)], axis=1)


def _k3_body(x3_ref, batch_ref,
             fc1w_ref, fc1b_ref, fc2w_ref, fc2b_ref, fc3w_ref, fc3b_ref,
             out_ref):
    x3 = x3_ref[...]
    batch = batch_ref[...]
    gids = lax.broadcasted_iota(jnp.int32, (1, NG), 1)
    onehot = (batch == gids).astype(jnp.float32)
    s = lax.dot_general(onehot, x3, (((0,), (0,)), ((), ())),
                        preferred_element_type=jnp.float32)
    ones_col = jnp.ones((NN, 1), jnp.float32)
    cnt = lax.dot_general(onehot, ones_col, (((0,), (0,)), ((), ())),
                          preferred_element_type=jnp.float32)
    mean = s / jnp.maximum(cnt, 1.0)
    emb = jnp.concatenate([mean[:NUM_GRAPHS], mean[NUM_GRAPHS:]], axis=1)
    hh = jax.nn.relu(jnp.dot(emb, fc1w_ref[...],
                             preferred_element_type=jnp.float32) + fc1b_ref[...])
    hh = jax.nn.relu(jnp.dot(hh, fc2w_ref[...],
                             preferred_element_type=jnp.float32) + fc2b_ref[...])
    out_ref[...] = jnp.dot(hh, fc3w_ref[...],
                           preferred_element_type=jnp.float32) + fc3b_ref[...]


def kernel(block_x, tx_x, b_W1, b_b1, b_g1, b_be1, b_W2, b_b2, b_g2, b_be2,
           t_W1, t_b1, t_g1, t_be1, t_W2, t_b2, t_g2, t_be2,
           fc1_W, fc1_b, fc2_W, fc2_b, fc3_W, fc3_b,
           block_edge_index, block_batch, tx_edge_index, tx_batch):
    f32 = jnp.float32
    src = jnp.concatenate([block_edge_index[0], tx_edge_index[0] + N_NODES])
    dst = jnp.concatenate([block_edge_index[1], tx_edge_index[1] + N_NODES])
    batch2d = jnp.concatenate([block_batch,
                               tx_batch + NUM_GRAPHS]).reshape(NN, 1)

    deg_parts = _deg_kernel(dst)

    hs1, dinv = pl.pallas_call(
        _k1_body,
        out_shape=(jax.ShapeDtypeStruct((NN, 2 * HIDDEN), f32),
                   jax.ShapeDtypeStruct((NN, HIDDEN), f32)),
    )(block_x, tx_x, b_W1, t_W1, deg_parts)

    acc1 = _scat_kernel(hs1, src, dst)

    x2 = pl.pallas_call(
        _k2a_body,
        out_shape=jax.ShapeDtypeStruct((NN, HIDDEN), f32),
    )(acc1, hs1, dinv, b_b1[None, :], t_b1[None, :], b_g1[None, :],
      t_g1[None, :], b_be1[None, :], t_be1[None, :])

    hs2 = pl.pallas_call(
        _k2b_body,
        out_shape=jax.ShapeDtypeStruct((NN, 2 * HIDDEN), f32),
    )(x2, b_W2, t_W2, dinv)

    acc2 = _scat_kernel(hs2, src, dst)

    x3 = pl.pallas_call(
        _k2a_body,
        out_shape=jax.ShapeDtypeStruct((NN, HIDDEN), f32),
    )(acc2, hs2, dinv, b_b2[None, :], t_b2[None, :], b_g2[None, :],
      t_g2[None, :], b_be2[None, :], t_be2[None, :])

    out = pl.pallas_call(
        _k3_body,
        out_shape=jax.ShapeDtypeStruct((NUM_GRAPHS, 2), f32),
    )(x3, batch2d,
      fc1_W, fc1_b[None, :], fc2_W, fc2_b[None, :], fc3_W, fc3_b[None, :])
    return out


# speedup vs baseline: 9.8318x; 1.5059x over previous
import functools

import jax
import jax.numpy as jnp
from jax import lax
from jax.experimental import pallas as pl
from jax.experimental.pallas import tpu as pltpu
from jax.experimental.pallas import tpu_sc as plsc

N_NODES = 10000
NUM_GRAPHS = 32
HIDDEN = 64
LW = 128

NN = 2 * N_NODES
NE = 2 * 320000
NG = 2 * NUM_GRAPHS

NC = 2
NS = 16
HALF = 10240
TRASH = HALF
ACCR = HALF + LW
RPT = ACCR // NS
WPT = HALF // NS

EPT = NE // NS
CH = 128
NFULL = EPT // CH
TAIL = EPT - NFULL * CH
SCK = 24
NSC = NFULL // SCK
SCE = SCK * CH
NBUF = 2

_mesh = plsc.VectorSubcoreMesh(core_axis_name="c", subcore_axis_name="s",
                               num_cores=NC, num_subcores=NS)


def _fill_rows(ref, nrows, value):
    vec = jnp.full((16,), value, jnp.float32)

    def body(r, _):
        for cc in range(LW // 16):
            ref[r, pl.ds(cc * 16, 16)] = vec
        return 0

    lax.fori_loop(0, nrows, body, 0)


def _zero_accum(rows_v, acc_sh, s):
    row0 = s * RPT
    for j in range(RPT // CH):
        pltpu.sync_copy(rows_v, acc_sh.at[pl.ds(row0 + j * CH, CH)])
    rem = RPT - (RPT // CH) * CH
    if rem:
        pltpu.sync_copy(rows_v.at[pl.ds(0, rem)],
                        acc_sh.at[pl.ds(row0 + (RPT // CH) * CH, rem)])


def _localize(dst_v, off, dstl_v, n, base):
    for j in range(n // 16):
        d = dst_v[pl.ds(off + j * 16, 16)] - base
        ok = (d >= 0) & (d < HALF)
        dstl_v[pl.ds(j * 16, 16)] = jnp.where(ok, d, TRASH)


def _writeout(acc_sh, out_hbm, c, s):
    for j in range(WPT // CH):
        r = s * WPT + j * CH
        pltpu.sync_copy(acc_sh.at[pl.ds(r, CH)], out_hbm.at[c, pl.ds(r, CH)])


@functools.partial(
    pl.kernel,
    out_type=jax.ShapeDtypeStruct((NC, HALF, LW), jnp.float32),
    mesh=_mesh,
    scratch_types=[
        pltpu.VMEM((SCE,), jnp.int32),
        pltpu.VMEM((SCE,), jnp.int32),
        pltpu.VMEM((SCK, CH), jnp.int32),
        pltpu.VMEM((TAIL,), jnp.int32),
        pltpu.VMEM((TAIL,), jnp.int32),
        pltpu.VMEM((NBUF, CH, LW), jnp.float32),
        pltpu.VMEM_SHARED((ACCR, LW), jnp.float32),
        [pltpu.SemaphoreType.DMA] * NBUF,
        [pltpu.SemaphoreType.DMA] * NBUF,
    ],
)
def _scat_kernel(hs_hbm, src_hbm, dst_hbm, out_hbm,
                 src_kv, dstg_v, dstl_kv, src_t, dstl_t, rows_v, acc_sh,
                 gsems, ssems):
    c = lax.axis_index("c")
    s = lax.axis_index("s")
    base = c * HALF
    _fill_rows(rows_v.at[0], CH, 0.0)
    _zero_accum(rows_v.at[0], acc_sh, s)
    plsc.subcore_barrier()

    ebase = s * EPT

    def body(k, _):
        off = pl.multiple_of(ebase + k * SCE, 8)
        pltpu.sync_copy(src_hbm.at[pl.ds(off, SCE)], src_kv)
        pltpu.sync_copy(dst_hbm.at[pl.ds(off, SCE)], dstg_v)
        for j in range(SCK):
            _localize(dstg_v, j * CH, dstl_kv.at[j], CH, base)
        gdesc = [None] * NBUF
        sdesc = [None] * NBUF
        for j in range(SCK):
            b = j % NBUF
            if sdesc[b] is not None:
                sdesc[b].wait()
            gdesc[b] = pltpu.async_copy(
                hs_hbm.at[src_kv.at[pl.ds(j * CH, CH)]], rows_v.at[b],
                gsems[b])
            if j > 0:
                pb = (j - 1) % NBUF
                gdesc[pb].wait()
                sdesc[pb] = pltpu.async_copy(
                    rows_v.at[pb], acc_sh.at[dstl_kv.at[j - 1]], ssems[pb],
                    add=True)
        lb = (SCK - 1) % NBUF
        gdesc[lb].wait()
        sdesc[lb] = pltpu.async_copy(
            rows_v.at[lb], acc_sh.at[dstl_kv.at[SCK - 1]], ssems[lb],
            add=True)
        for b in range(NBUF):
            sdesc[b].wait()
        return 0

    lax.fori_loop(0, NSC, body, 0)

    offt = pl.multiple_of(ebase + NFULL * CH, 8)
    pltpu.sync_copy(src_hbm.at[pl.ds(offt, TAIL)], src_t)
    pltpu.sync_copy(dst_hbm.at[pl.ds(offt, TAIL)], dstg_v.at[pl.ds(0, TAIL)])
    _localize(dstg_v, 0, dstl_t, TAIL, base)
    pltpu.async_copy(hs_hbm.at[src_t], rows_v.at[0, pl.ds(0, TAIL)],
                     gsems[0]).wait()
    pltpu.sync_copy(rows_v.at[0, pl.ds(0, TAIL)], acc_sh.at[dstl_t], add=True)

    plsc.subcore_barrier()
    _writeout(acc_sh, out_hbm, c, s)


@functools.partial(
    pl.kernel,
    out_type=jax.ShapeDtypeStruct((NC, HALF, LW), jnp.float32),
    mesh=_mesh,
    scratch_types=[
        pltpu.VMEM((CH,), jnp.int32),
        pltpu.VMEM((TAIL,), jnp.int32),
        pltpu.VMEM((CH,), jnp.int32),
        pltpu.VMEM((TAIL,), jnp.int32),
        pltpu.VMEM((CH, LW), jnp.float32),
        pltpu.VMEM((CH, LW), jnp.float32),
        pltpu.VMEM_SHARED((ACCR, LW), jnp.float32),
    ],
)
def _deg_kernel(dst_hbm, out_hbm, dst_v, dst_t, dstl_v, dstl_t, zer_v, ones_v,
                acc_sh):
    c = lax.axis_index("c")
    s = lax.axis_index("s")
    base = c * HALF
    _fill_rows(zer_v, CH, 0.0)
    _fill_rows(ones_v, CH, 1.0)
    _zero_accum(zer_v, acc_sh, s)
    plsc.subcore_barrier()

    ebase = s * EPT

    def body(i, _):
        off = pl.multiple_of(ebase + i * CH, 8)
        pltpu.sync_copy(dst_hbm.at[pl.ds(off, CH)], dst_v)
        _localize(dst_v, 0, dstl_v, CH, base)
        pltpu.sync_copy(ones_v, acc_sh.at[dstl_v], add=True)
        return 0

    lax.fori_loop(0, NFULL, body, 0)

    offt = pl.multiple_of(ebase + NFULL * CH, 8)
    pltpu.sync_copy(dst_hbm.at[pl.ds(offt, TAIL)], dst_t)
    _localize(dst_t, 0, dstl_t, TAIL, base)
    pltpu.sync_copy(ones_v.at[pl.ds(0, TAIL)], acc_sh.at[dstl_t], add=True)

    plsc.subcore_barrier()
    _writeout(acc_sh, out_hbm, c, s)


_DOT = dict(preferred_element_type=jnp.float32,
            precision=lax.Precision.DEFAULT)


def _unsplit(parts_ref):
    full = jnp.concatenate([parts_ref[0], parts_ref[1]], axis=0)
    return full[:NN, :HIDDEN]


def _kd_body(degp_ref, dinv_ref):
    dinv_ref[...] = lax.rsqrt(_unsplit(degp_ref) + 1.0)


def _k1_body(bx_ref, tx_ref, bw_ref, tw_ref, dinv_ref, hs_ref):
    hb = jnp.dot(bx_ref[...], bw_ref[...], **_DOT)
    ht = jnp.dot(tx_ref[...], tw_ref[...], **_DOT)
    h = jnp.concatenate([hb, ht], axis=0)
    hs_ref[:, :HIDDEN] = h * dinv_ref[...]
    hs_ref[:, HIDDEN:] = jnp.zeros((NN, HIDDEN), jnp.float32)


def _combine_bn_relu(acc_ref, hs_ref, dinv_ref, b_b, t_b, b_g, t_g, b_be, t_be):
    acc = _unsplit(acc_ref)
    dinv = dinv_ref[...]
    hs = hs_ref[...][:, :HIDDEN]
    z = dinv * acc + dinv * hs
    zb = z[:N_NODES] + b_b[...]
    zt = z[N_NODES:] + t_b[...]

    def bn_relu(y, g, be):
        m = jnp.mean(y, axis=0, keepdims=True)
        v = jnp.mean((y - m) * (y - m), axis=0, keepdims=True)
        return jax.nn.relu((y - m) * lax.rsqrt(v + 1e-5) * g + be)

    return bn_relu(zb, b_g[...], b_be[...]), bn_relu(zt, t_g[...], t_be[...])


def _k2a_body(acc_ref, hs_ref, dinv_ref, bb_ref, tb_ref, bg_ref, tg_ref,
              bbe_ref, tbe_ref, x2_ref):
    xb, xt = _combine_bn_relu(acc_ref, hs_ref, dinv_ref, bb_ref, tb_ref,
                              bg_ref, tg_ref, bbe_ref, tbe_ref)
    x2_ref[...] = jnp.concatenate([xb, xt], axis=0)


def _k2b_body(x2_ref, bw2_ref, tw2_ref, dinv_ref, hs2_ref):
    x2 = x2_ref[...]
    hb = jnp.dot(x2[:N_NODES], bw2_ref[...], **_DOT)
    ht = jnp.dot(x2[N_NODES:], tw2_ref[...], **_DOT)
    h2 = jnp.concatenate([hb, ht], axis=0)
    hs2_ref[:, :HIDDEN] = h2 * dinv_ref[...]
    hs2_ref[:, HIDDEN:] = jnp.zeros((NN, HIDDEN), jnp.float32)


def _k3_body(x3_ref, batch_ref,
             fc1w_ref, fc1b_ref, fc2w_ref, fc2b_ref, fc3w_ref, fc3b_ref,
             out_ref):
    x3 = x3_ref[...]
    batch = batch_ref[...]
    gids = lax.broadcasted_iota(jnp.int32, (1, NG), 1)
    onehot = (batch == gids).astype(jnp.float32)
    s = lax.dot_general(onehot, x3, (((0,), (0,)), ((), ())),
                        preferred_element_type=jnp.float32,
                        precision=lax.Precision.HIGHEST)
    ones_col = jnp.ones((NN, 1), jnp.float32)
    cnt = lax.dot_general(onehot, ones_col, (((0,), (0,)), ((), ())),
                          preferred_element_type=jnp.float32,
                          precision=lax.Precision.HIGHEST)
    mean = s / jnp.maximum(cnt, 1.0)
    emb = jnp.concatenate([mean[:NUM_GRAPHS], mean[NUM_GRAPHS:]], axis=1)
    hh = jax.nn.relu(jnp.dot(emb, fc1w_ref[...], **_DOT) + fc1b_ref[...])
    hh = jax.nn.relu(jnp.dot(hh, fc2w_ref[...], **_DOT) + fc2b_ref[...])
    out_ref[...] = jnp.dot(hh, fc3w_ref[...], **_DOT) + fc3b_ref[...]


def kernel(block_x, tx_x, b_W1, b_b1, b_g1, b_be1, b_W2, b_b2, b_g2, b_be2,
           t_W1, t_b1, t_g1, t_be1, t_W2, t_b2, t_g2, t_be2,
           fc1_W, fc1_b, fc2_W, fc2_b, fc3_W, fc3_b,
           block_edge_index, block_batch, tx_edge_index, tx_batch):
    f32 = jnp.float32
    src = jnp.concatenate([block_edge_index[0], tx_edge_index[0] + N_NODES])
    dst = jnp.concatenate([block_edge_index[1], tx_edge_index[1] + N_NODES])
    batch2d = jnp.concatenate([block_batch,
                               tx_batch + NUM_GRAPHS]).reshape(NN, 1)

    deg_parts = _deg_kernel(dst)

    dinv = pl.pallas_call(
        _kd_body,
        out_shape=jax.ShapeDtypeStruct((NN, HIDDEN), f32),
    )(deg_parts)

    hs1 = pl.pallas_call(
        _k1_body,
        out_shape=jax.ShapeDtypeStruct((NN, 2 * HIDDEN), f32),
    )(block_x, tx_x, b_W1, t_W1, dinv)

    acc1 = _scat_kernel(hs1, src, dst)

    x2 = pl.pallas_call(
        _k2a_body,
        out_shape=jax.ShapeDtypeStruct((NN, HIDDEN), f32),
    )(acc1, hs1, dinv, b_b1[None, :], t_b1[None, :], b_g1[None, :],
      t_g1[None, :], b_be1[None, :], t_be1[None, :])

    hs2 = pl.pallas_call(
        _k2b_body,
        out_shape=jax.ShapeDtypeStruct((NN, 2 * HIDDEN), f32),
    )(x2, b_W2, t_W2, dinv)

    acc2 = _scat_kernel(hs2, src, dst)

    x3 = pl.pallas_call(
        _k2a_body,
        out_shape=jax.ShapeDtypeStruct((NN, HIDDEN), f32),
    )(acc2, hs2, dinv, b_b2[None, :], t_b2[None, :], b_g2[None, :],
      t_g2[None, :], b_be2[None, :], t_be2[None, :])

    out = pl.pallas_call(
        _k3_body,
        out_shape=jax.ShapeDtypeStruct((NUM_GRAPHS, 2), f32),
    )(x3, batch2d,
      fc1_W, fc1_b[None, :], fc2_W, fc2_b[None, :], fc3_W, fc3_b[None, :])
    return out
